# baseline (speedup 1.0000x reference)
import functools

import jax
import jax.numpy as jnp
from jax import lax
from jax.experimental import pallas as pl
from jax.experimental.pallas import tpu as pltpu
from jax.experimental.pallas import tpu_sc as plsc

BATCH = 16384
EMBED = 64
HIDDEN = 256

NC = 2
NS = 16
NW = NC * NS
BPW = BATCH // NW
CHUNK = 128
NCH = BPW // CHUNK

_mesh = plsc.VectorSubcoreMesh(core_axis_name="c", subcore_axis_name="s")


@functools.partial(
    pl.kernel,
    mesh=_mesh,
    out_type=[
        jax.ShapeDtypeStruct((BATCH, EMBED), jnp.float32),
        jax.ShapeDtypeStruct((BATCH, EMBED), jnp.float32),
    ],
    scratch_types=[
        pltpu.VMEM((NCH, CHUNK), jnp.int32),
        pltpu.VMEM((NCH, CHUNK), jnp.int32),
        pltpu.VMEM((BPW, EMBED), jnp.float32),
        pltpu.VMEM((BPW, EMBED), jnp.float32),
        pltpu.SemaphoreType.DMA,
    ],
    compiler_params=pltpu.CompilerParams(use_tc_tiling_on_sc=False),
)
def _gather_sc(users_hbm, books_hbm, utab_hbm, btab_hbm, ue_hbm, be_hbm,
               uidx, bidx, urows, brows, sem):
    wid = lax.axis_index("s") * NC + lax.axis_index("c")
    base = wid * BPW
    pltpu.sync_copy(users_hbm.at[pl.ds(wid * NCH, NCH)], uidx)
    pltpu.sync_copy(books_hbm.at[pl.ds(wid * NCH, NCH)], bidx)
    copies = []
    for j in range(NCH):
        copies.append(pltpu.async_copy(
            utab_hbm.at[uidx.at[j]], urows.at[pl.ds(j * CHUNK, CHUNK)], sem))
        copies.append(pltpu.async_copy(
            btab_hbm.at[bidx.at[j]], brows.at[pl.ds(j * CHUNK, CHUNK)], sem))
    for c in copies:
        c.wait()
    pltpu.sync_copy(urows, ue_hbm.at[pl.ds(base, BPW)])
    pltpu.sync_copy(brows, be_hbm.at[pl.ds(base, BPW)])


BM = 2048


def _mlp_body(ue_ref, be_ref, w1u_ref, w1b_ref, b1_ref, w2r_ref, b2_ref,
              out_ref):
    h = jnp.dot(ue_ref[...], w1u_ref[...], preferred_element_type=jnp.float32)
    h = h + jnp.dot(be_ref[...], w1b_ref[...],
                    preferred_element_type=jnp.float32)
    h = jnp.maximum(h + b1_ref[...], 0.0)
    out_ref[...] = (jnp.sum(h * w2r_ref[...], axis=1, keepdims=True)
                    + b2_ref[...])


def _mlp_tc(ue, be, W1, b1, W2, b2):
    w1u = W1[:EMBED]
    w1b = W1[EMBED:]
    b1r = b1.reshape(1, HIDDEN)
    w2r = W2.reshape(1, HIDDEN)
    b2r = b2.reshape(1, 1)
    grid = (BATCH // BM,)
    return pl.pallas_call(
        _mlp_body,
        grid=grid,
        in_specs=[
            pl.BlockSpec((BM, EMBED), lambda i: (i, 0)),
            pl.BlockSpec((BM, EMBED), lambda i: (i, 0)),
            pl.BlockSpec((EMBED, HIDDEN), lambda i: (0, 0)),
            pl.BlockSpec((EMBED, HIDDEN), lambda i: (0, 0)),
            pl.BlockSpec((1, HIDDEN), lambda i: (0, 0)),
            pl.BlockSpec((1, HIDDEN), lambda i: (0, 0)),
            pl.BlockSpec((1, 1), lambda i: (0, 0)),
        ],
        out_specs=pl.BlockSpec((BM, 1), lambda i: (i, 0)),
        out_shape=jax.ShapeDtypeStruct((BATCH, 1), jnp.float32),
    )(ue, be, w1u, w1b, b1r, w2r, b2r)


def kernel(users, books, user_table, book_table, W1, b1, W2, b2):
    users2d = users.astype(jnp.int32).reshape(NW * NCH, CHUNK)
    books2d = books.astype(jnp.int32).reshape(NW * NCH, CHUNK)
    ue, be = _gather_sc(users2d, books2d, user_table, book_table)
    return _mlp_tc(ue, be, W1, b1, W2, b2)


# speedup vs baseline: 1.7284x; 1.7284x over previous
import functools

import jax
import jax.numpy as jnp
from jax import lax
from jax.experimental import pallas as pl
from jax.experimental.pallas import tpu as pltpu
from jax.experimental.pallas import tpu_sc as plsc

V = 1000000
EMBED = 64
BATCH = 16384
HIDDEN = 256

NC = 2
NS = 16
NW = NC * NS
BPW = 244
C = 4
R = BPW // C
TAIL_START = NW * BPW * 128
TAIL = V - TAIL_START
CAP = 640
LISTC = 1024
NLV = LISTC // 16
SENT = 1 << 29
NOUT = BATCH + 128
WORDS = EMBED // 2

_mesh = plsc.VectorSubcoreMesh(core_axis_name="c", subcore_axis_name="s")

_IOTA = lambda: lax.iota(jnp.int32, 16)


@functools.partial(
    pl.kernel,
    mesh=_mesh,
    out_type=[
        jax.ShapeDtypeStruct((NW, 2, CAP // 32, 8, 128), jnp.int32),
        jax.ShapeDtypeStruct((NW, 2, 8, 128), jnp.int32),
    ],
    scratch_types=[
        pltpu.VMEM((BATCH,), jnp.int32),
        pltpu.VMEM((2, C, 8, 8, 128), jnp.float32),
        pltpu.VMEM((LISTC,), jnp.int32),
        pltpu.VMEM((LISTC,), jnp.int32),
        pltpu.VMEM((32,), jnp.int32),
        pltpu.VMEM((32,), jnp.int32),
        pltpu.VMEM((CAP // 32, 8, 128), jnp.int32),
        pltpu.VMEM((8, 128), jnp.int32),
        pltpu.SemaphoreType.DMA,
    ],
    compiler_params=pltpu.CompilerParams(use_tc_tiling_on_sc=True,
                                         needs_layout_passes=False),
)
def _scan_sc(users_hbm, books_hbm, t3u_hbm, t3b_hbm, vals_hbm, pos_hbm,
             idxvm, chunk, i_list, n_list, rb_i, rb_g, staging, pos2d, sem):
    wid = lax.axis_index("s") * NC + lax.axis_index("c")
    wlo = wid * BPW

    for t in range(2):
        idx_hbm = users_hbm if t == 0 else books_hbm
        tbl = t3u_hbm if t == 0 else t3b_hbm

        def fire(rr, slot, tbl=tbl):
            for c in range(C):
                off = pl.multiple_of((wlo + rr * C + c) * 128, 128)
                pltpu.async_copy(tbl.at[:, :, pl.ds(off, 128)],
                                 chunk.at[slot, c], sem)

        def drain_round(slot, tbl=tbl):
            for c in range(C):
                pltpu.make_async_copy(tbl.at[:, :, pl.ds(0, 128)],
                                      chunk.at[slot, c], sem).wait()

        pltpu.sync_copy(idx_hbm, idxvm)

        def initv(u):
            s16 = jnp.full((16,), SENT, jnp.int32)
            i_list[pl.ds(u * 16, 16)] = s16
            n_list[pl.ds(u * 16, 16)] = s16

        pl.loop(0, NLV)(initv)

        for u in range(8):
            for mm in range(8):
                d = BATCH + ((wid * 16 + u * 8 + mm + _IOTA()) & 127)
                pos2d[u, pl.ds(mm * 16, 16)] = d

        def prevec(u, cnt):
            iv = idxvm[pl.ds(u * 16, 16)]
            q = lax.shift_right_logical(iv, 7)
            m = (q >= wlo) & (q < wlo + BPW)
            mi = m.astype(jnp.int32)
            npop = plsc.all_reduce_population_count(m)[0]
            slot = jnp.minimum(cnt + plsc.cumsum(mi) - mi, LISTC - 1)
            nv = u * 16 + _IOTA()
            plsc.store_scatter(i_list, [slot], iv, mask=m)
            plsc.store_scatter(n_list, [slot], nv, mask=m)
            return cnt + npop

        lax.fori_loop(0, BATCH // 16, prevec, jnp.int32(0))

        fire(0, 0)

        def round_body(r, gcnt):
            ring = r & 1

            @pl.when(r + 1 < R)
            def _():
                fire(r + 1, (r + 1) & 1)

            drain_round(ring)

            qlo = wlo + r * C

            def rscan(u, rcnt):
                liv = i_list[pl.ds(u * 16, 16)]
                lnv = n_list[pl.ds(u * 16, 16)]
                q = lax.shift_right_logical(liv, 7)
                m = (q >= qlo) & (q < qlo + C)
                mi = m.astype(jnp.int32)
                npop = plsc.all_reduce_population_count(m)[0]
                sir = rcnt + plsc.cumsum(mi) - mi
                rslot = jnp.minimum(sir, 31)
                gslot = jnp.minimum(gcnt + sir, CAP - 1)
                plsc.store_scatter(rb_i, [rslot], liv, mask=m)
                plsc.store_scatter(rb_g, [rslot], gslot, mask=m)
                plsc.store_scatter(pos2d,
                                   [lax.shift_right_logical(gslot, 7),
                                    gslot & 127], lnv, mask=m)
                return rcnt + npop

            rcnt = lax.fori_loop(0, NLV, rscan, jnp.int32(0))

            for g in range(2):
                @pl.when(rcnt > g * 16)
                def _(g=g):
                    mg = (g * 16 + _IOTA()) < rcnt
                    vi = rb_i[pl.ds(g * 16, 16)]
                    vg = rb_g[pl.ds(g * 16, 16)]
                    cvec = (lax.shift_right_logical(vi, 7) - qlo) & (C - 1)
                    lvec = vi & 127
                    rsp = jnp.full((16,), ring, jnp.int32)
                    for j in range(0, EMBED, 2):
                        jt = jnp.full((16,), j // 8, jnp.int32)
                        jra = jnp.full((16,), j % 8, jnp.int32)
                        jrb = jnp.full((16,), j % 8 + 1, jnp.int32)
                        ga = plsc.load_gather(chunk, [rsp, cvec, jt, jra, lvec],
                                              mask=mg)
                        gb = plsc.load_gather(chunk, [rsp, cvec, jt, jrb, lvec],
                                              mask=mg)
                        pk = plsc.pack(ga, gb, format=plsc.PackFormat.INTERLEAVED)
                        w32 = plsc.bitcast(pk, jnp.int32)
                        W = vg * WORDS + (j >> 1)
                        plsc.store_scatter(
                            staging,
                            [lax.shift_right_logical(W, 10),
                             lax.shift_right_logical(W, 7) & 7,
                             W & 127], w32, mask=mg)

            return gcnt + rcnt

        lax.fori_loop(0, R, round_body, jnp.int32(0))

        pltpu.sync_copy(staging, vals_hbm.at[wid, t])
        pltpu.sync_copy(pos2d, pos_hbm.at[wid, t])


@functools.partial(
    pl.kernel,
    mesh=_mesh,
    out_type=[
        jax.ShapeDtypeStruct((NOUT, WORDS), jnp.int32),
        jax.ShapeDtypeStruct((NOUT, WORDS), jnp.int32),
    ],
    scratch_types=[
        pltpu.VMEM((CAP, WORDS), jnp.int32),
        pltpu.VMEM((8, 128), jnp.int32),
        pltpu.VMEM((4, 128), jnp.int32),
        pltpu.VMEM((4, 128), jnp.int32),
        pltpu.VMEM((512, WORDS), jnp.int32),
        pltpu.SemaphoreType.DMA,
    ],
    compiler_params=pltpu.CompilerParams(use_tc_tiling_on_sc=False),
)
def _scatter_sc(vals_hbm, pos_hbm, users2d_hbm, books2d_hbm,
                tailu_hbm, tailb_hbm, ue_hbm, be_hbm,
                valsvm, posvm, gi2d, tpos2d, ttvm, sem):
    wid = lax.axis_index("s") * NC + lax.axis_index("c")

    for t in range(2):
        out = ue_hbm if t == 0 else be_hbm
        tail_tbl = tailu_hbm if t == 0 else tailb_hbm
        idx2d = users2d_hbm if t == 0 else books2d_hbm

        pltpu.sync_copy(vals_hbm.at[wid, t], valsvm)
        pltpu.sync_copy(pos_hbm.at[wid, t], posvm)

        copies = []
        for b in range(CAP // 128):
            copies.append(pltpu.async_copy(
                valsvm.at[pl.ds(b * 128, 128)], out.at[posvm.at[b]], sem))
        for cp in copies:
            cp.wait()

        pltpu.sync_copy(idx2d.at[pl.ds(wid * 4, 4)], gi2d)

        def tvec(a):
            def tv16(mm):
                iv = gi2d[a, pl.ds(mm * 16, 16)]
                m = iv >= TAIL_START
                n0 = wid * 512 + a * 128 + mm * 16 + _IOTA()
                dump = BATCH + ((n0 + wid) & 127)
                tpos2d[a, pl.ds(mm * 16, 16)] = jnp.where(m, n0, dump)
                gi2d[a, pl.ds(mm * 16, 16)] = jnp.where(
                    m, iv - TAIL_START, 0)
            for mm in range(8):
                tv16(mm)

        for a in range(4):
            tvec(a)

        copies = []
        for a in range(4):
            copies.append(pltpu.async_copy(
                tail_tbl.at[gi2d.at[a]], ttvm.at[pl.ds(a * 128, 128)], sem))
        for cp in copies:
            cp.wait()
        copies = []
        for a in range(4):
            copies.append(pltpu.async_copy(
                ttvm.at[pl.ds(a * 128, 128)], out.at[tpos2d.at[a]], sem))
        for cp in copies:
            cp.wait()


BM = 2048


def _mlp_body(ue_ref, be_ref, w1u_ref, w1b_ref, b1_ref, w2r_ref, b2_ref,
              out_ref):
    h = jnp.dot(ue_ref[...], w1u_ref[...], preferred_element_type=jnp.float32)
    h = h + jnp.dot(be_ref[...], w1b_ref[...],
                    preferred_element_type=jnp.float32)
    h = jnp.maximum(h + b1_ref[...], 0.0)
    out_ref[...] = (jnp.sum(h * w2r_ref[...], axis=1, keepdims=True)
                    + b2_ref[...])


def _mlp_tc(ue, be, W1, b1, W2, b2):
    w1u = W1[:EMBED].astype(jnp.bfloat16)
    w1b = W1[EMBED:].astype(jnp.bfloat16)
    b1r = b1.reshape(1, HIDDEN)
    w2r = W2.reshape(1, HIDDEN)
    b2r = b2.reshape(1, 1)
    grid = (BATCH // BM,)
    return pl.pallas_call(
        _mlp_body,
        grid=grid,
        in_specs=[
            pl.BlockSpec((BM, EMBED), lambda i: (i, 0)),
            pl.BlockSpec((BM, EMBED), lambda i: (i, 0)),
            pl.BlockSpec((EMBED, HIDDEN), lambda i: (0, 0)),
            pl.BlockSpec((EMBED, HIDDEN), lambda i: (0, 0)),
            pl.BlockSpec((1, HIDDEN), lambda i: (0, 0)),
            pl.BlockSpec((1, HIDDEN), lambda i: (0, 0)),
            pl.BlockSpec((1, 1), lambda i: (0, 0)),
        ],
        out_specs=pl.BlockSpec((BM, 1), lambda i: (i, 0)),
        out_shape=jax.ShapeDtypeStruct((BATCH, 1), jnp.float32),
    )(ue, be, w1u, w1b, b1r, w2r, b2r)


def _pack_tail(tail_f32):
    tb = tail_f32.astype(jnp.bfloat16).reshape(TAIL, WORDS, 2)
    return lax.bitcast_convert_type(tb, jnp.int32).reshape(TAIL, WORDS)


def kernel(users, books, user_table, book_table, W1, b1, W2, b2):
    users_i = users.astype(jnp.int32)
    books_i = books.astype(jnp.int32)
    t3u = user_table.T.reshape(8, 8, V)
    t3b = book_table.T.reshape(8, 8, V)

    vals, pos = _scan_sc(users_i, books_i, t3u, t3b)
    vals2 = vals.reshape(NW, 2, CAP, WORDS)

    tailu = _pack_tail(user_table[TAIL_START:])
    tailb = _pack_tail(book_table[TAIL_START:])
    u2d = users_i.reshape(128, 128)
    b2d = books_i.reshape(128, 128)

    ueP, beP = _scatter_sc(vals2, pos, u2d, b2d, tailu, tailb)

    ue = lax.bitcast_convert_type(ueP, jnp.bfloat16).reshape(NOUT, EMBED)
    be = lax.bitcast_convert_type(beP, jnp.bfloat16).reshape(NOUT, EMBED)
    return _mlp_tc(ue[:BATCH], be[:BATCH], W1, b1, W2, b2)


# speedup vs baseline: 3.2395x; 1.8743x over previous
import functools

import jax
import jax.numpy as jnp
from jax import lax
from jax.experimental import pallas as pl
from jax.experimental.pallas import tpu as pltpu
from jax.experimental.pallas import tpu_sc as plsc

V = 1000000
EMBED = 64
BATCH = 16384
HIDDEN = 256

NC = 2
NS = 16
NW = NC * NS
BPW = 244
C = 4
R = BPW // C
TAIL_START = NW * BPW * 128
TAIL = V - TAIL_START
CAP = 640
LISTC = 1024
NLV = LISTC // 16
SENT = 1 << 29
NOUT = BATCH + 128
WORDS = EMBED // 2

_mesh = plsc.VectorSubcoreMesh(core_axis_name="c", subcore_axis_name="s")

_IOTA = lambda: lax.iota(jnp.int32, 16)


@functools.partial(
    pl.kernel,
    mesh=_mesh,
    out_type=[
        jax.ShapeDtypeStruct((NW, 2, CAP // 32, 8, 128), jnp.int32),
        jax.ShapeDtypeStruct((NW, 2, 8, 128), jnp.int32),
    ],
    scratch_types=[
        pltpu.VMEM((BATCH,), jnp.int32),
        pltpu.VMEM((2, C, 8, 8, 128), jnp.float32),
        pltpu.VMEM((LISTC,), jnp.int32),
        pltpu.VMEM((LISTC,), jnp.int32),
        pltpu.VMEM((32,), jnp.int32),
        pltpu.VMEM((32,), jnp.int32),
        pltpu.VMEM((CAP // 32, 8, 128), jnp.int32),
        pltpu.VMEM((8, 128), jnp.int32),
        pltpu.SemaphoreType.DMA,
    ],
    compiler_params=pltpu.CompilerParams(use_tc_tiling_on_sc=True,
                                         needs_layout_passes=False),
)
def _scan_sc(users_hbm, books_hbm, t3u_hbm, t3b_hbm, vals_hbm, pos_hbm,
             idxvm, chunk, i_list, n_list, rb_i, rb_g, staging, pos2d, sem):
    wid = lax.axis_index("s") * NC + lax.axis_index("c")
    wlo = wid * BPW

    for t in range(2):
        idx_hbm = users_hbm if t == 0 else books_hbm
        tbl = t3u_hbm if t == 0 else t3b_hbm

        def fire(rr, slot, tbl=tbl):
            for c in range(C):
                off = pl.multiple_of((wlo + rr * C + c) * 128, 128)
                pltpu.async_copy(tbl.at[:, :, pl.ds(off, 128)],
                                 chunk.at[slot, c], sem)

        def drain_round(slot, tbl=tbl):
            for c in range(C):
                pltpu.make_async_copy(tbl.at[:, :, pl.ds(0, 128)],
                                      chunk.at[slot, c], sem).wait()

        pltpu.sync_copy(idx_hbm, idxvm)

        def initv(u):
            s16 = jnp.full((16,), SENT, jnp.int32)
            i_list[pl.ds(u * 16, 16)] = s16
            n_list[pl.ds(u * 16, 16)] = s16

        pl.loop(0, NLV)(initv)

        for u in range(8):
            for mm in range(8):
                d = BATCH + ((wid * 16 + u * 8 + mm + _IOTA()) & 127)
                pos2d[u, pl.ds(mm * 16, 16)] = d

        def prevec(u, cnt):
            iv = idxvm[pl.ds(u * 16, 16)]
            q = lax.shift_right_logical(iv, 7)
            m = (q >= wlo) & (q < wlo + BPW)
            mi = m.astype(jnp.int32)
            npop = plsc.all_reduce_population_count(m)[0]
            slot = jnp.minimum(cnt + plsc.cumsum(mi) - mi, LISTC - 1)
            nv = u * 16 + _IOTA()
            plsc.store_scatter(i_list, [slot], iv, mask=m)
            plsc.store_scatter(n_list, [slot], nv, mask=m)
            return cnt + npop

        cnt = lax.fori_loop(0, BATCH // 16, prevec, jnp.int32(0))
        nlv = lax.shift_right_logical(
            jnp.minimum(cnt, LISTC) + 15, 4)

        fire(0, 0)

        def round_body(r, gcnt):
            ring = r & 1

            @pl.when(r + 1 < R)
            def _():
                fire(r + 1, (r + 1) & 1)

            drain_round(ring)

            qlo = wlo + r * C

            def rscan(u, rcnt):
                liv = i_list[pl.ds(u * 16, 16)]
                lnv = n_list[pl.ds(u * 16, 16)]
                q = lax.shift_right_logical(liv, 7)
                m = (q >= qlo) & (q < qlo + C)
                mi = m.astype(jnp.int32)
                npop = plsc.all_reduce_population_count(m)[0]
                sir = rcnt + plsc.cumsum(mi) - mi
                rslot = jnp.minimum(sir, 31)
                gslot = jnp.minimum(gcnt + sir, CAP - 1)
                plsc.store_scatter(rb_i, [rslot], liv, mask=m)
                plsc.store_scatter(rb_g, [rslot], gslot, mask=m)
                plsc.store_scatter(pos2d,
                                   [lax.shift_right_logical(gslot, 7),
                                    gslot & 127], lnv, mask=m)
                return rcnt + npop

            rcnt = lax.fori_loop(0, nlv, rscan, jnp.int32(0))

            for g in range(2):
                @pl.when(rcnt > g * 16)
                def _(g=g):
                    mg = (g * 16 + _IOTA()) < rcnt
                    vi = rb_i[pl.ds(g * 16, 16)]
                    vg = rb_g[pl.ds(g * 16, 16)]
                    cvec = (lax.shift_right_logical(vi, 7) - qlo) & (C - 1)
                    lvec = vi & 127
                    rsp = jnp.full((16,), ring, jnp.int32)
                    for j in range(0, EMBED, 2):
                        jt = jnp.full((16,), j // 8, jnp.int32)
                        jra = jnp.full((16,), j % 8, jnp.int32)
                        jrb = jnp.full((16,), j % 8 + 1, jnp.int32)
                        ga = plsc.load_gather(chunk, [rsp, cvec, jt, jra, lvec],
                                              mask=mg)
                        gb = plsc.load_gather(chunk, [rsp, cvec, jt, jrb, lvec],
                                              mask=mg)
                        pk = plsc.pack(ga, gb, format=plsc.PackFormat.INTERLEAVED)
                        w32 = plsc.bitcast(pk, jnp.int32)
                        W = vg * WORDS + (j >> 1)
                        plsc.store_scatter(
                            staging,
                            [lax.shift_right_logical(W, 10),
                             lax.shift_right_logical(W, 7) & 7,
                             W & 127], w32, mask=mg)

            return gcnt + rcnt

        lax.fori_loop(0, R, round_body, jnp.int32(0))

        pltpu.sync_copy(staging, vals_hbm.at[wid, t])
        pltpu.sync_copy(pos2d, pos_hbm.at[wid, t])


@functools.partial(
    pl.kernel,
    mesh=_mesh,
    out_type=[
        jax.ShapeDtypeStruct((NOUT, WORDS), jnp.int32),
        jax.ShapeDtypeStruct((NOUT, WORDS), jnp.int32),
    ],
    scratch_types=[
        pltpu.VMEM((CAP, WORDS), jnp.int32),
        pltpu.VMEM((8, 128), jnp.int32),
        pltpu.VMEM((4, 128), jnp.int32),
        pltpu.VMEM((4, 128), jnp.int32),
        pltpu.VMEM((512, WORDS), jnp.int32),
        pltpu.SemaphoreType.DMA,
    ],
    compiler_params=pltpu.CompilerParams(use_tc_tiling_on_sc=False),
)
def _scatter_sc(vals_hbm, pos_hbm, users2d_hbm, books2d_hbm,
                tailu_hbm, tailb_hbm, ue_hbm, be_hbm,
                valsvm, posvm, gi2d, tpos2d, ttvm, sem):
    wid = lax.axis_index("s") * NC + lax.axis_index("c")

    for t in range(2):
        out = ue_hbm if t == 0 else be_hbm
        tail_tbl = tailu_hbm if t == 0 else tailb_hbm
        idx2d = users2d_hbm if t == 0 else books2d_hbm

        pltpu.sync_copy(vals_hbm.at[wid, t], valsvm)
        pltpu.sync_copy(pos_hbm.at[wid, t], posvm)

        copies = []
        for b in range(CAP // 128):
            copies.append(pltpu.async_copy(
                valsvm.at[pl.ds(b * 128, 128)], out.at[posvm.at[b]], sem))
        for cp in copies:
            cp.wait()

        pltpu.sync_copy(idx2d.at[pl.ds(wid * 4, 4)], gi2d)

        def tvec(a):
            def tv16(mm):
                iv = gi2d[a, pl.ds(mm * 16, 16)]
                m = iv >= TAIL_START
                n0 = wid * 512 + a * 128 + mm * 16 + _IOTA()
                dump = BATCH + ((n0 + wid) & 127)
                tpos2d[a, pl.ds(mm * 16, 16)] = jnp.where(m, n0, dump)
                gi2d[a, pl.ds(mm * 16, 16)] = jnp.where(
                    m, iv - TAIL_START, n0 & 511)
            for mm in range(8):
                tv16(mm)

        for a in range(4):
            tvec(a)

        copies = []
        for a in range(4):
            copies.append(pltpu.async_copy(
                tail_tbl.at[gi2d.at[a]], ttvm.at[pl.ds(a * 128, 128)], sem))
        for cp in copies:
            cp.wait()
        copies = []
        for a in range(4):
            copies.append(pltpu.async_copy(
                ttvm.at[pl.ds(a * 128, 128)], out.at[tpos2d.at[a]], sem))
        for cp in copies:
            cp.wait()


BM = 2048


def _mlp_body(ue_ref, be_ref, w1u_ref, w1b_ref, b1_ref, w2r_ref, b2_ref,
              out_ref):
    h = jnp.dot(ue_ref[...], w1u_ref[...], preferred_element_type=jnp.float32)
    h = h + jnp.dot(be_ref[...], w1b_ref[...],
                    preferred_element_type=jnp.float32)
    h = jnp.maximum(h + b1_ref[...], 0.0)
    out_ref[...] = (jnp.sum(h * w2r_ref[...], axis=1, keepdims=True)
                    + b2_ref[...])


def _mlp_tc(ue, be, W1, b1, W2, b2):
    w1u = W1[:EMBED].astype(jnp.bfloat16)
    w1b = W1[EMBED:].astype(jnp.bfloat16)
    b1r = b1.reshape(1, HIDDEN)
    w2r = W2.reshape(1, HIDDEN)
    b2r = b2.reshape(1, 1)
    grid = (BATCH // BM,)
    return pl.pallas_call(
        _mlp_body,
        grid=grid,
        in_specs=[
            pl.BlockSpec((BM, EMBED), lambda i: (i, 0)),
            pl.BlockSpec((BM, EMBED), lambda i: (i, 0)),
            pl.BlockSpec((EMBED, HIDDEN), lambda i: (0, 0)),
            pl.BlockSpec((EMBED, HIDDEN), lambda i: (0, 0)),
            pl.BlockSpec((1, HIDDEN), lambda i: (0, 0)),
            pl.BlockSpec((1, HIDDEN), lambda i: (0, 0)),
            pl.BlockSpec((1, 1), lambda i: (0, 0)),
        ],
        out_specs=pl.BlockSpec((BM, 1), lambda i: (i, 0)),
        out_shape=jax.ShapeDtypeStruct((BATCH, 1), jnp.float32),
    )(ue, be, w1u, w1b, b1r, w2r, b2r)


def _pack_tail(tail_f32):
    tb = tail_f32.astype(jnp.bfloat16).reshape(TAIL, WORDS, 2)
    return lax.bitcast_convert_type(tb, jnp.int32).reshape(TAIL, WORDS)


def kernel(users, books, user_table, book_table, W1, b1, W2, b2):
    users_i = users.astype(jnp.int32)
    books_i = books.astype(jnp.int32)
    t3u = user_table.T.reshape(8, 8, V)
    t3b = book_table.T.reshape(8, 8, V)

    vals, pos = _scan_sc(users_i, books_i, t3u, t3b)
    vals2 = vals.reshape(NW, 2, CAP, WORDS)

    tailu = _pack_tail(user_table[TAIL_START:])
    tailb = _pack_tail(book_table[TAIL_START:])
    u2d = users_i.reshape(128, 128)
    b2d = books_i.reshape(128, 128)

    ueP, beP = _scatter_sc(vals2, pos, u2d, b2d, tailu, tailb)

    ue = lax.bitcast_convert_type(ueP, jnp.bfloat16).reshape(NOUT, EMBED)
    be = lax.bitcast_convert_type(beP, jnp.bfloat16).reshape(NOUT, EMBED)
    return _mlp_tc(ue[:BATCH], be[:BATCH], W1, b1, W2, b2)


# speedup vs baseline: 3.2479x; 1.0026x over previous
import functools

import jax
import jax.numpy as jnp
from jax import lax
from jax.experimental import pallas as pl
from jax.experimental.pallas import tpu as pltpu
from jax.experimental.pallas import tpu_sc as plsc

V = 1000000
EMBED = 64
BATCH = 16384
HIDDEN = 256

NC = 2
NS = 16
NW = NC * NS
BPW = 244
C = 4
R = BPW // C
TAIL_START = NW * BPW * 128
TAIL = V - TAIL_START
CAP = 640
LISTC = 1024
NLV = LISTC // 16
SENT = 1 << 29
NOUT = BATCH + 128
WORDS = EMBED // 2

_mesh = plsc.VectorSubcoreMesh(core_axis_name="c", subcore_axis_name="s")

_IOTA = lambda: lax.iota(jnp.int32, 16)


@functools.partial(
    pl.kernel,
    mesh=_mesh,
    out_type=[
        jax.ShapeDtypeStruct((NW, 2, CAP // 32, 8, 128), jnp.int32),
        jax.ShapeDtypeStruct((NW, 2, 8, 128), jnp.int32),
    ],
    scratch_types=[
        pltpu.VMEM((BATCH,), jnp.int32),
        pltpu.VMEM((2, 8, 8, C * 128), jnp.float32),
        pltpu.VMEM((LISTC,), jnp.int32),
        pltpu.VMEM((LISTC,), jnp.int32),
        pltpu.VMEM((32,), jnp.int32),
        pltpu.VMEM((32,), jnp.int32),
        pltpu.VMEM((CAP // 32, 8, 128), jnp.int32),
        pltpu.VMEM((8, 128), jnp.int32),
        pltpu.SemaphoreType.DMA,
    ],
    compiler_params=pltpu.CompilerParams(use_tc_tiling_on_sc=True,
                                         needs_layout_passes=False),
)
def _scan_sc(users_hbm, books_hbm, t3u_hbm, t3b_hbm, vals_hbm, pos_hbm,
             idxvm, chunk, i_list, n_list, rb_i, rb_g, staging, pos2d, sem):
    wid = lax.axis_index("s") * NC + lax.axis_index("c")
    wlo = wid * BPW

    for t in range(2):
        idx_hbm = users_hbm if t == 0 else books_hbm
        tbl = t3u_hbm if t == 0 else t3b_hbm

        def fire(rr, slot, tbl=tbl):
            off = pl.multiple_of((wlo + rr * C) * 128, 128)
            for jt in range(8):
                pltpu.async_copy(tbl.at[jt, :, pl.ds(off, C * 128)],
                                 chunk.at[slot, jt], sem)

        def drain_round(slot, tbl=tbl):
            for jt in range(8):
                pltpu.make_async_copy(tbl.at[0, :, pl.ds(0, C * 128)],
                                      chunk.at[slot, jt], sem).wait()

        pltpu.sync_copy(idx_hbm, idxvm)

        def initv(u):
            s16 = jnp.full((16,), SENT, jnp.int32)
            i_list[pl.ds(u * 16, 16)] = s16
            n_list[pl.ds(u * 16, 16)] = s16

        pl.loop(0, NLV)(initv)

        for u in range(8):
            for mm in range(8):
                d = BATCH + ((wid * 16 + u * 8 + mm + _IOTA()) & 127)
                pos2d[u, pl.ds(mm * 16, 16)] = d

        def prevec(u, cnt):
            iv = idxvm[pl.ds(u * 16, 16)]
            q = lax.shift_right_logical(iv, 7)
            m = (q >= wlo) & (q < wlo + BPW)
            mi = m.astype(jnp.int32)
            npop = plsc.all_reduce_population_count(m)[0]
            slot = jnp.minimum(cnt + plsc.cumsum(mi) - mi, LISTC - 1)
            nv = u * 16 + _IOTA()
            plsc.store_scatter(i_list, [slot], iv, mask=m)
            plsc.store_scatter(n_list, [slot], nv, mask=m)
            return cnt + npop

        cnt = lax.fori_loop(0, BATCH // 16, prevec, jnp.int32(0))
        nlv = lax.shift_right_logical(
            jnp.minimum(cnt, LISTC) + 15, 4)

        fire(0, 0)

        def round_body(r, gcnt):
            ring = r & 1

            @pl.when(r + 1 < R)
            def _():
                fire(r + 1, (r + 1) & 1)

            drain_round(ring)

            qlo = wlo + r * C

            def rscan(u, rcnt):
                liv = i_list[pl.ds(u * 16, 16)]
                lnv = n_list[pl.ds(u * 16, 16)]
                q = lax.shift_right_logical(liv, 7)
                m = (q >= qlo) & (q < qlo + C)
                mi = m.astype(jnp.int32)
                npop = plsc.all_reduce_population_count(m)[0]
                sir = rcnt + plsc.cumsum(mi) - mi
                rslot = jnp.minimum(sir, 31)
                gslot = jnp.minimum(gcnt + sir, CAP - 1)
                plsc.store_scatter(rb_i, [rslot], liv, mask=m)
                plsc.store_scatter(rb_g, [rslot], gslot, mask=m)
                plsc.store_scatter(pos2d,
                                   [lax.shift_right_logical(gslot, 7),
                                    gslot & 127], lnv, mask=m)
                return rcnt + npop

            rcnt = lax.fori_loop(0, nlv, rscan, jnp.int32(0))

            for g in range(2):
                @pl.when(rcnt > g * 16)
                def _(g=g):
                    mg = (g * 16 + _IOTA()) < rcnt
                    vi = rb_i[pl.ds(g * 16, 16)]
                    vg = rb_g[pl.ds(g * 16, 16)]
                    cvec = (lax.shift_right_logical(vi, 7) - qlo) & (C - 1)
                    lvec = vi & 127
                    rsp = jnp.full((16,), ring, jnp.int32)
                    col = cvec * 128 + lvec
                    for j in range(0, EMBED, 2):
                        jt = jnp.full((16,), j // 8, jnp.int32)
                        jra = jnp.full((16,), j % 8, jnp.int32)
                        jrb = jnp.full((16,), j % 8 + 1, jnp.int32)
                        ga = plsc.load_gather(chunk, [rsp, jt, jra, col],
                                              mask=mg)
                        gb = plsc.load_gather(chunk, [rsp, jt, jrb, col],
                                              mask=mg)
                        pk = plsc.pack(ga, gb, format=plsc.PackFormat.INTERLEAVED)
                        w32 = plsc.bitcast(pk, jnp.int32)
                        W = vg * WORDS + (j >> 1)
                        plsc.store_scatter(
                            staging,
                            [lax.shift_right_logical(W, 10),
                             lax.shift_right_logical(W, 7) & 7,
                             W & 127], w32, mask=mg)

            return gcnt + rcnt

        lax.fori_loop(0, R, round_body, jnp.int32(0))

        pltpu.sync_copy(staging, vals_hbm.at[wid, t])
        pltpu.sync_copy(pos2d, pos_hbm.at[wid, t])


@functools.partial(
    pl.kernel,
    mesh=_mesh,
    out_type=[
        jax.ShapeDtypeStruct((NOUT, WORDS), jnp.int32),
        jax.ShapeDtypeStruct((NOUT, WORDS), jnp.int32),
    ],
    scratch_types=[
        pltpu.VMEM((CAP, WORDS), jnp.int32),
        pltpu.VMEM((8, 128), jnp.int32),
        pltpu.VMEM((4, 128), jnp.int32),
        pltpu.VMEM((4, 128), jnp.int32),
        pltpu.VMEM((512, WORDS), jnp.int32),
        pltpu.SemaphoreType.DMA,
    ],
    compiler_params=pltpu.CompilerParams(use_tc_tiling_on_sc=False),
)
def _scatter_sc(vals_hbm, pos_hbm, users2d_hbm, books2d_hbm,
                tailu_hbm, tailb_hbm, ue_hbm, be_hbm,
                valsvm, posvm, gi2d, tpos2d, ttvm, sem):
    wid = lax.axis_index("s") * NC + lax.axis_index("c")

    for t in range(2):
        out = ue_hbm if t == 0 else be_hbm
        tail_tbl = tailu_hbm if t == 0 else tailb_hbm
        idx2d = users2d_hbm if t == 0 else books2d_hbm

        pltpu.sync_copy(vals_hbm.at[wid, t], valsvm)
        pltpu.sync_copy(pos_hbm.at[wid, t], posvm)

        copies = []
        for b in range(CAP // 128):
            copies.append(pltpu.async_copy(
                valsvm.at[pl.ds(b * 128, 128)], out.at[posvm.at[b]], sem))
        for cp in copies:
            cp.wait()

        pltpu.sync_copy(idx2d.at[pl.ds(wid * 4, 4)], gi2d)

        def tvec(a):
            def tv16(mm):
                iv = gi2d[a, pl.ds(mm * 16, 16)]
                m = iv >= TAIL_START
                n0 = wid * 512 + a * 128 + mm * 16 + _IOTA()
                dump = BATCH + ((n0 + wid) & 127)
                tpos2d[a, pl.ds(mm * 16, 16)] = jnp.where(m, n0, dump)
                gi2d[a, pl.ds(mm * 16, 16)] = jnp.where(
                    m, iv - TAIL_START, n0 & 511)
            for mm in range(8):
                tv16(mm)

        for a in range(4):
            tvec(a)

        copies = []
        for a in range(4):
            copies.append(pltpu.async_copy(
                tail_tbl.at[gi2d.at[a]], ttvm.at[pl.ds(a * 128, 128)], sem))
        for cp in copies:
            cp.wait()
        copies = []
        for a in range(4):
            copies.append(pltpu.async_copy(
                ttvm.at[pl.ds(a * 128, 128)], out.at[tpos2d.at[a]], sem))
        for cp in copies:
            cp.wait()


BM = 2048


def _mlp_body(ue_ref, be_ref, w1u_ref, w1b_ref, b1_ref, w2r_ref, b2_ref,
              out_ref):
    h = jnp.dot(ue_ref[...], w1u_ref[...], preferred_element_type=jnp.float32)
    h = h + jnp.dot(be_ref[...], w1b_ref[...],
                    preferred_element_type=jnp.float32)
    h = jnp.maximum(h + b1_ref[...], 0.0)
    out_ref[...] = (jnp.sum(h * w2r_ref[...], axis=1, keepdims=True)
                    + b2_ref[...])


def _mlp_tc(ue, be, W1, b1, W2, b2):
    w1u = W1[:EMBED].astype(jnp.bfloat16)
    w1b = W1[EMBED:].astype(jnp.bfloat16)
    b1r = b1.reshape(1, HIDDEN)
    w2r = W2.reshape(1, HIDDEN)
    b2r = b2.reshape(1, 1)
    grid = (BATCH // BM,)
    return pl.pallas_call(
        _mlp_body,
        grid=grid,
        in_specs=[
            pl.BlockSpec((BM, EMBED), lambda i: (i, 0)),
            pl.BlockSpec((BM, EMBED), lambda i: (i, 0)),
            pl.BlockSpec((EMBED, HIDDEN), lambda i: (0, 0)),
            pl.BlockSpec((EMBED, HIDDEN), lambda i: (0, 0)),
            pl.BlockSpec((1, HIDDEN), lambda i: (0, 0)),
            pl.BlockSpec((1, HIDDEN), lambda i: (0, 0)),
            pl.BlockSpec((1, 1), lambda i: (0, 0)),
        ],
        out_specs=pl.BlockSpec((BM, 1), lambda i: (i, 0)),
        out_shape=jax.ShapeDtypeStruct((BATCH, 1), jnp.float32),
    )(ue, be, w1u, w1b, b1r, w2r, b2r)


def _pack_tail(tail_f32):
    tb = tail_f32.astype(jnp.bfloat16).reshape(TAIL, WORDS, 2)
    return lax.bitcast_convert_type(tb, jnp.int32).reshape(TAIL, WORDS)


def kernel(users, books, user_table, book_table, W1, b1, W2, b2):
    users_i = users.astype(jnp.int32)
    books_i = books.astype(jnp.int32)
    t3u = user_table.T.reshape(8, 8, V)
    t3b = book_table.T.reshape(8, 8, V)

    vals, pos = _scan_sc(users_i, books_i, t3u, t3b)
    vals2 = vals.reshape(NW, 2, CAP, WORDS)

    tailu = _pack_tail(user_table[TAIL_START:])
    tailb = _pack_tail(book_table[TAIL_START:])
    u2d = users_i.reshape(128, 128)
    b2d = books_i.reshape(128, 128)

    ueP, beP = _scatter_sc(vals2, pos, u2d, b2d, tailu, tailb)

    ue = lax.bitcast_convert_type(ueP, jnp.bfloat16).reshape(NOUT, EMBED)
    be = lax.bitcast_convert_type(beP, jnp.bfloat16).reshape(NOUT, EMBED)
    return _mlp_tc(ue[:BATCH], be[:BATCH], W1, b1, W2, b2)


# speedup vs baseline: 3.9518x; 1.2167x over previous
import functools

import jax
import jax.numpy as jnp
from jax import lax
from jax.experimental import pallas as pl
from jax.experimental.pallas import tpu as pltpu
from jax.experimental.pallas import tpu_sc as plsc

V = 1000000
EMBED = 64
BATCH = 16384
HIDDEN = 256

NC = 2
NS = 16
NW = NC * NS
BPW = 244
C = 4
R = BPW // C
TAIL_START = NW * BPW * 128
TAIL = V - TAIL_START
CAP = 640
LISTC = 1024
NLV = LISTC // 16
SENT = 1 << 29
NOUT = BATCH + 128
WORDS = EMBED // 2

_mesh = plsc.VectorSubcoreMesh(core_axis_name="c", subcore_axis_name="s")

_IOTA = lambda: lax.iota(jnp.int32, 16)


@functools.partial(
    pl.kernel,
    mesh=_mesh,
    out_type=[
        jax.ShapeDtypeStruct((NW, 2, CAP // 32, 8, 128), jnp.int32),
        jax.ShapeDtypeStruct((NW, 2, 8, 128), jnp.int32),
    ],
    scratch_types=[
        pltpu.VMEM((BATCH,), jnp.int32),
        pltpu.VMEM((2, 8, 8, C * 128), jnp.float32),
        pltpu.VMEM((LISTC,), jnp.int32),
        pltpu.VMEM((LISTC,), jnp.int32),
        pltpu.VMEM((32,), jnp.int32),
        pltpu.VMEM((32,), jnp.int32),
        pltpu.VMEM((CAP // 32, 8, 128), jnp.int32),
        pltpu.VMEM((8, 128), jnp.int32),
        pltpu.SemaphoreType.DMA,
    ],
    compiler_params=pltpu.CompilerParams(use_tc_tiling_on_sc=True,
                                         needs_layout_passes=False),
)
def _scan_sc(users_hbm, books_hbm, t3u_hbm, t3b_hbm, vals_hbm, pos_hbm,
             idxvm, chunk, i_list, n_list, rb_i, rb_g, staging, pos2d, sem):
    wid = lax.axis_index("s") * NC + lax.axis_index("c")
    wlo = wid * BPW

    for t in range(2):
        idx_hbm = users_hbm if t == 0 else books_hbm
        tbl = t3u_hbm if t == 0 else t3b_hbm

        def fire(rr, slot, tbl=tbl):
            off = pl.multiple_of((wlo + rr * C) * 128, 128)
            for jt in range(8):
                pltpu.async_copy(tbl.at[jt, :, pl.ds(off, C * 128)],
                                 chunk.at[slot, jt], sem)

        def drain_round(slot, tbl=tbl):
            for jt in range(8):
                pltpu.make_async_copy(tbl.at[0, :, pl.ds(0, C * 128)],
                                      chunk.at[slot, jt], sem).wait()

        pltpu.sync_copy(idx_hbm, idxvm)

        def initv(u):
            s16 = jnp.full((16,), SENT, jnp.int32)
            i_list[pl.ds(u * 16, 16)] = s16
            n_list[pl.ds(u * 16, 16)] = s16

        pl.loop(0, NLV)(initv)

        for u in range(8):
            for mm in range(8):
                d = BATCH + ((wid * 16 + u * 8 + mm + _IOTA()) & 127)
                pos2d[u, pl.ds(mm * 16, 16)] = d

        def prevec(u, cnt):
            iv = idxvm[pl.ds(u * 16, 16)]
            q = lax.shift_right_logical(iv, 7)
            m = (q >= wlo) & (q < wlo + BPW)
            mi = m.astype(jnp.int32)
            npop = plsc.all_reduce_population_count(m)[0]
            slot = jnp.minimum(cnt + plsc.cumsum(mi) - mi, LISTC - 1)
            nv = u * 16 + _IOTA()
            plsc.store_scatter(i_list, [slot], iv, mask=m)
            plsc.store_scatter(n_list, [slot], nv, mask=m)
            return cnt + npop

        cnt = lax.fori_loop(0, BATCH // 16, prevec, jnp.int32(0))
        nlv = lax.shift_right_logical(
            jnp.minimum(cnt, LISTC) + 15, 4)

        fire(0, 0)

        def round_body(r, gcnt):
            ring = r & 1

            @pl.when(r + 1 < R)
            def _():
                fire(r + 1, (r + 1) & 1)

            drain_round(ring)

            qlo = wlo + r * C

            def rscan(u, rcnt):
                liv = i_list[pl.ds(u * 16, 16)]
                lnv = n_list[pl.ds(u * 16, 16)]
                q = lax.shift_right_logical(liv, 7)
                m = (q >= qlo) & (q < qlo + C)
                mi = m.astype(jnp.int32)
                npop = plsc.all_reduce_population_count(m)[0]
                sir = rcnt + plsc.cumsum(mi) - mi
                rslot = jnp.minimum(sir, 31)
                gslot = jnp.minimum(gcnt + sir, CAP - 1)
                plsc.store_scatter(rb_i, [rslot], liv, mask=m)
                plsc.store_scatter(rb_g, [rslot], gslot, mask=m)
                plsc.store_scatter(pos2d,
                                   [lax.shift_right_logical(gslot, 7),
                                    gslot & 127], lnv, mask=m)
                return rcnt + npop

            rcnt = lax.fori_loop(0, nlv, rscan, jnp.int32(0))

            for g in range(2):
                @pl.when(rcnt > g * 16)
                def _(g=g):
                    mg = (g * 16 + _IOTA()) < rcnt
                    vi = rb_i[pl.ds(g * 16, 16)]
                    vg = rb_g[pl.ds(g * 16, 16)]
                    cvec = (lax.shift_right_logical(vi, 7) - qlo) & (C - 1)
                    lvec = vi & 127
                    rsp = jnp.full((16,), ring, jnp.int32)
                    col = cvec * 128 + lvec
                    for j in range(0, EMBED, 2):
                        jt = jnp.full((16,), j // 8, jnp.int32)
                        jra = jnp.full((16,), j % 8, jnp.int32)
                        jrb = jnp.full((16,), j % 8 + 1, jnp.int32)
                        ga = plsc.load_gather(chunk, [rsp, jt, jra, col],
                                              mask=mg)
                        gb = plsc.load_gather(chunk, [rsp, jt, jrb, col],
                                              mask=mg)
                        pk = plsc.pack(ga, gb, format=plsc.PackFormat.INTERLEAVED)
                        w32 = plsc.bitcast(pk, jnp.int32)
                        W = vg * WORDS + (j >> 1)
                        plsc.store_scatter(
                            staging,
                            [lax.shift_right_logical(W, 10),
                             lax.shift_right_logical(W, 7) & 7,
                             W & 127], w32, mask=mg)

            return gcnt + rcnt

        lax.fori_loop(0, R, round_body, jnp.int32(0))

        pltpu.sync_copy(staging, vals_hbm.at[wid, t])
        pltpu.sync_copy(pos2d, pos_hbm.at[wid, t])


@functools.partial(
    pl.kernel,
    mesh=_mesh,
    out_type=[
        jax.ShapeDtypeStruct((NOUT, WORDS), jnp.int32),
        jax.ShapeDtypeStruct((NOUT, WORDS), jnp.int32),
    ],
    scratch_types=[
        pltpu.VMEM((CAP, WORDS), jnp.int32),
        pltpu.VMEM((8, 128), jnp.int32),
        pltpu.VMEM((4, 128), jnp.int32),
        pltpu.VMEM((4, 128), jnp.int32),
        pltpu.VMEM((512, WORDS), jnp.int32),
        pltpu.SemaphoreType.DMA,
    ],
    compiler_params=pltpu.CompilerParams(use_tc_tiling_on_sc=False),
)
def _scatter_sc(vals_hbm, pos_hbm, users2d_hbm, books2d_hbm,
                tailu_hbm, tailb_hbm, ue_hbm, be_hbm,
                valsvm, posvm, gi2d, tpos2d, ttvm, sem):
    wid = lax.axis_index("s") * NC + lax.axis_index("c")

    for t in range(2):
        out = ue_hbm if t == 0 else be_hbm
        tail_tbl = tailu_hbm if t == 0 else tailb_hbm
        idx2d = users2d_hbm if t == 0 else books2d_hbm

        pltpu.sync_copy(vals_hbm.at[wid, t], valsvm)
        pltpu.sync_copy(pos_hbm.at[wid, t], posvm)

        copies = []
        for b in range(CAP // 128):
            copies.append(pltpu.async_copy(
                valsvm.at[pl.ds(b * 128, 128)], out.at[posvm.at[b]], sem))
        for cp in copies:
            cp.wait()

        pltpu.sync_copy(idx2d.at[pl.ds(wid * 4, 4)], gi2d)

        def tvec(a):
            def tv16(mm):
                iv = gi2d[a, pl.ds(mm * 16, 16)]
                m = iv >= TAIL_START
                n0 = wid * 512 + a * 128 + mm * 16 + _IOTA()
                dump = BATCH + ((n0 + wid) & 127)
                tpos2d[a, pl.ds(mm * 16, 16)] = jnp.where(m, n0, dump)
                gi2d[a, pl.ds(mm * 16, 16)] = jnp.where(
                    m, iv - TAIL_START, n0 & 511)
            for mm in range(8):
                tv16(mm)

        for a in range(4):
            tvec(a)

        copies = []
        for a in range(4):
            copies.append(pltpu.async_copy(
                tail_tbl.at[gi2d.at[a]], ttvm.at[pl.ds(a * 128, 128)], sem))
        for cp in copies:
            cp.wait()
        copies = []
        for a in range(4):
            copies.append(pltpu.async_copy(
                ttvm.at[pl.ds(a * 128, 128)], out.at[tpos2d.at[a]], sem))
        for cp in copies:
            cp.wait()


BM4 = 512
QH = 4 * HIDDEN


def _unpack_lohi(x):
    lo = lax.bitcast_convert_type((x & 0xFFFF).astype(jnp.uint16),
                                  jnp.bfloat16)
    hi = lax.bitcast_convert_type(
        lax.shift_right_logical(x, 16).astype(jnp.uint16), jnp.bfloat16)
    return lo, hi


def _mlp_body(ue_ref, be_ref, w1ue_ref, w1uo_ref, w1be_ref, w1bo_ref,
              b1_ref, w2r_ref, b2_ref, out_ref):
    ulo, uhi = _unpack_lohi(ue_ref[...])
    blo, bhi = _unpack_lohi(be_ref[...])
    h = jnp.dot(ulo, w1ue_ref[...], preferred_element_type=jnp.float32)
    h += jnp.dot(uhi, w1uo_ref[...], preferred_element_type=jnp.float32)
    h += jnp.dot(blo, w1be_ref[...], preferred_element_type=jnp.float32)
    h += jnp.dot(bhi, w1bo_ref[...], preferred_element_type=jnp.float32)
    h = jnp.maximum(h + b1_ref[...], 0.0)
    hw = h * w2r_ref[...]
    segs = [jnp.sum(hw[:, k * HIDDEN:(k + 1) * HIDDEN], axis=1,
                    keepdims=True) for k in range(4)]
    out_ref[...] = jnp.concatenate(segs, axis=1) + b2_ref[...]


def _quad_eo(w_bf):
    z = jnp.zeros_like(w_bf)
    rows = [jnp.concatenate([w_bf if i == k else z for i in range(4)], axis=1)
            for k in range(4)]
    bd = jnp.concatenate(rows, axis=0)
    return bd[0::2], bd[1::2]


def _mlp_tc(uePr, bePr, W1, b1, W2, b2):
    w1ue, w1uo = _quad_eo(W1[:EMBED].astype(jnp.bfloat16))
    w1be, w1bo = _quad_eo(W1[EMBED:].astype(jnp.bfloat16))
    b1t = jnp.tile(b1, 4).reshape(1, QH)
    w2t = jnp.tile(W2.reshape(-1), 4).reshape(1, QH)
    b2r = b2.reshape(1, 1)
    grid = (BATCH // (4 * BM4),)
    out4 = pl.pallas_call(
        _mlp_body,
        grid=grid,
        in_specs=[
            pl.BlockSpec((BM4, 128), lambda i: (i, 0)),
            pl.BlockSpec((BM4, 128), lambda i: (i, 0)),
            pl.BlockSpec((128, QH), lambda i: (0, 0)),
            pl.BlockSpec((128, QH), lambda i: (0, 0)),
            pl.BlockSpec((128, QH), lambda i: (0, 0)),
            pl.BlockSpec((128, QH), lambda i: (0, 0)),
            pl.BlockSpec((1, QH), lambda i: (0, 0)),
            pl.BlockSpec((1, QH), lambda i: (0, 0)),
            pl.BlockSpec((1, 1), lambda i: (0, 0)),
        ],
        out_specs=pl.BlockSpec((BM4, 4), lambda i: (i, 0)),
        out_shape=jax.ShapeDtypeStruct((BATCH // 4, 4), jnp.float32),
    )(uePr, bePr, w1ue, w1uo, w1be, w1bo, b1t, w2t, b2r)
    return out4.reshape(BATCH, 1)


def _pack_tail(tail_f32):
    tb = tail_f32.astype(jnp.bfloat16).reshape(TAIL, WORDS, 2)
    return lax.bitcast_convert_type(tb, jnp.int32).reshape(TAIL, WORDS)


def kernel(users, books, user_table, book_table, W1, b1, W2, b2):
    users_i = users.astype(jnp.int32)
    books_i = books.astype(jnp.int32)
    t3u = user_table.T.reshape(8, 8, V)
    t3b = book_table.T.reshape(8, 8, V)

    vals, pos = _scan_sc(users_i, books_i, t3u, t3b)
    vals2 = vals.reshape(NW, 2, CAP, WORDS)

    tailu = _pack_tail(user_table[TAIL_START:])
    tailb = _pack_tail(book_table[TAIL_START:])
    u2d = users_i.reshape(128, 128)
    b2d = books_i.reshape(128, 128)

    ueP, beP = _scatter_sc(vals2, pos, u2d, b2d, tailu, tailb)

    uePr = ueP.reshape(NOUT * WORDS // 128, 128)
    bePr = beP.reshape(NOUT * WORDS // 128, 128)
    return _mlp_tc(uePr, bePr, W1, b1, W2, b2)


# speedup vs baseline: 4.0802x; 1.0325x over previous
import functools

import jax
import jax.numpy as jnp
from jax import lax
from jax.experimental import pallas as pl
from jax.experimental.pallas import tpu as pltpu
from jax.experimental.pallas import tpu_sc as plsc

V = 1000000
EMBED = 64
BATCH = 16384
HIDDEN = 256

NC = 2
NS = 16
NW = NC * NS
BPW = 244
C = 4
R = BPW // C
TAIL_START = NW * BPW * 128
TAIL = V - TAIL_START
CAP = 640
LISTC = 1024
NLV = LISTC // 16
SENT = 1 << 29
NOUT = BATCH + 128
WORDS = EMBED // 2

_mesh = plsc.VectorSubcoreMesh(core_axis_name="c", subcore_axis_name="s")

_IOTA = lambda: lax.iota(jnp.int32, 16)


@functools.partial(
    pl.kernel,
    mesh=_mesh,
    out_type=[
        jax.ShapeDtypeStruct((NW, 2, CAP // 32, 8, 128), jnp.int32),
        jax.ShapeDtypeStruct((NW, 2, 8, 128), jnp.int32),
    ],
    scratch_types=[
        pltpu.VMEM((BATCH,), jnp.int32),
        pltpu.VMEM((2, C, 8, 8, 128), jnp.float32),
        pltpu.VMEM((256,), jnp.int32),
        pltpu.VMEM((LISTC,), jnp.int32),
        pltpu.VMEM((LISTC,), jnp.int32),
        pltpu.VMEM((32,), jnp.int32),
        pltpu.VMEM((32,), jnp.int32),
        pltpu.VMEM((CAP // 32, 8, 128), jnp.int32),
        pltpu.VMEM((8, 128), jnp.int32),
        pltpu.SemaphoreType.DMA,
    ],
    compiler_params=pltpu.CompilerParams(use_tc_tiling_on_sc=True,
                                         needs_layout_passes=False),
)
def _scan_sc(users_hbm, books_hbm, t3u_hbm, t3b_hbm, vals_hbm, pos_hbm,
             idxvm, chunk, occv, i_list, n_list, rb_i, rb_g, staging, pos2d,
             sem):
    wid = lax.axis_index("s") * NC + lax.axis_index("c")
    wlo = wid * BPW

    for t in range(2):
        idx_hbm = users_hbm if t == 0 else books_hbm
        tbl = t3u_hbm if t == 0 else t3b_hbm

        def rflags(rr):
            b0 = rr * C
            base = lax.shift_right_logical(b0, 4) * 16
            fvec = occv[pl.ds(base, 16)]
            return jnp.take(fvec, (b0 - base) + _IOTA())

        def fire(rr, slot, tbl=tbl):
            fl = rflags(rr)
            for c in range(C):
                @pl.when(fl[c] > 0)
                def _(c=c):
                    off = pl.multiple_of((wlo + rr * C + c) * 128, 128)
                    pltpu.async_copy(tbl.at[:, :, pl.ds(off, 128)],
                                     chunk.at[slot, c], sem)

        def drain_round(rr, slot, tbl=tbl):
            fl = rflags(rr)
            for c in range(C):
                @pl.when(fl[c] > 0)
                def _(c=c):
                    pltpu.make_async_copy(tbl.at[:, :, pl.ds(0, 128)],
                                          chunk.at[slot, c], sem).wait()

        pltpu.sync_copy(idx_hbm, idxvm)

        def initv(u):
            s16 = jnp.full((16,), SENT, jnp.int32)
            i_list[pl.ds(u * 16, 16)] = s16
            n_list[pl.ds(u * 16, 16)] = s16

        pl.loop(0, NLV)(initv)

        for u in range(8):
            for mm in range(8):
                d = BATCH + ((wid * 16 + u * 8 + mm + _IOTA()) & 127)
                pos2d[u, pl.ds(mm * 16, 16)] = d

        def initocc(u):
            occv[pl.ds(u * 16, 16)] = jnp.zeros((16,), jnp.int32)

        pl.loop(0, 16)(initocc)

        def prevec(u, cnt):
            iv = idxvm[pl.ds(u * 16, 16)]
            q = lax.shift_right_logical(iv, 7)
            m = (q >= wlo) & (q < wlo + BPW)
            mi = m.astype(jnp.int32)
            npop = plsc.all_reduce_population_count(m)[0]
            slot = jnp.minimum(cnt + plsc.cumsum(mi) - mi, LISTC - 1)
            nv = u * 16 + _IOTA()
            plsc.store_scatter(i_list, [slot], iv, mask=m)
            plsc.store_scatter(n_list, [slot], nv, mask=m)
            plsc.store_scatter(occv, [(q - wlo) & 255], mi, mask=m)
            return cnt + npop

        cnt = lax.fori_loop(0, BATCH // 16, prevec, jnp.int32(0))
        nlv = lax.shift_right_logical(
            jnp.minimum(cnt, LISTC) + 15, 4)

        fire(0, 0)

        def round_body(r, gcnt):
            ring = r & 1

            @pl.when(r + 1 < R)
            def _():
                fire(r + 1, (r + 1) & 1)

            drain_round(r, ring)

            qlo = wlo + r * C

            def rscan(u, rcnt):
                liv = i_list[pl.ds(u * 16, 16)]
                lnv = n_list[pl.ds(u * 16, 16)]
                q = lax.shift_right_logical(liv, 7)
                m = (q >= qlo) & (q < qlo + C)
                mi = m.astype(jnp.int32)
                npop = plsc.all_reduce_population_count(m)[0]
                sir = rcnt + plsc.cumsum(mi) - mi
                rslot = jnp.minimum(sir, 31)
                gslot = jnp.minimum(gcnt + sir, CAP - 1)
                plsc.store_scatter(rb_i, [rslot], liv, mask=m)
                plsc.store_scatter(rb_g, [rslot], gslot, mask=m)
                plsc.store_scatter(pos2d,
                                   [lax.shift_right_logical(gslot, 7),
                                    gslot & 127], lnv, mask=m)
                return rcnt + npop

            rcnt = lax.fori_loop(0, nlv, rscan, jnp.int32(0))

            for g in range(2):
                @pl.when(rcnt > g * 16)
                def _(g=g):
                    mg = (g * 16 + _IOTA()) < rcnt
                    vi = rb_i[pl.ds(g * 16, 16)]
                    vg = rb_g[pl.ds(g * 16, 16)]
                    cvec = (lax.shift_right_logical(vi, 7) - qlo) & (C - 1)
                    lvec = vi & 127
                    rsp = jnp.full((16,), ring, jnp.int32)
                    for j in range(0, EMBED, 2):
                        jt = jnp.full((16,), j // 8, jnp.int32)
                        jra = jnp.full((16,), j % 8, jnp.int32)
                        jrb = jnp.full((16,), j % 8 + 1, jnp.int32)
                        ga = plsc.load_gather(chunk, [rsp, cvec, jt, jra, lvec],
                                              mask=mg)
                        gb = plsc.load_gather(chunk, [rsp, cvec, jt, jrb, lvec],
                                              mask=mg)
                        pk = plsc.pack(ga, gb, format=plsc.PackFormat.INTERLEAVED)
                        w32 = plsc.bitcast(pk, jnp.int32)
                        W = vg * WORDS + (j >> 1)
                        plsc.store_scatter(
                            staging,
                            [lax.shift_right_logical(W, 10),
                             lax.shift_right_logical(W, 7) & 7,
                             W & 127], w32, mask=mg)

            return gcnt + rcnt

        lax.fori_loop(0, R, round_body, jnp.int32(0))

        pltpu.sync_copy(staging, vals_hbm.at[wid, t])
        pltpu.sync_copy(pos2d, pos_hbm.at[wid, t])


@functools.partial(
    pl.kernel,
    mesh=_mesh,
    out_type=[
        jax.ShapeDtypeStruct((NOUT, WORDS), jnp.int32),
        jax.ShapeDtypeStruct((NOUT, WORDS), jnp.int32),
    ],
    scratch_types=[
        pltpu.VMEM((CAP, WORDS), jnp.int32),
        pltpu.VMEM((8, 128), jnp.int32),
        pltpu.VMEM((4, 128), jnp.int32),
        pltpu.VMEM((4, 128), jnp.int32),
        pltpu.VMEM((512, WORDS), jnp.int32),
        pltpu.SemaphoreType.DMA,
    ],
    compiler_params=pltpu.CompilerParams(use_tc_tiling_on_sc=False),
)
def _scatter_sc(vals_hbm, pos_hbm, users2d_hbm, books2d_hbm,
                tailu_hbm, tailb_hbm, ue_hbm, be_hbm,
                valsvm, posvm, gi2d, tpos2d, ttvm, sem):
    wid = lax.axis_index("s") * NC + lax.axis_index("c")

    for t in range(2):
        out = ue_hbm if t == 0 else be_hbm
        tail_tbl = tailu_hbm if t == 0 else tailb_hbm
        idx2d = users2d_hbm if t == 0 else books2d_hbm

        pltpu.sync_copy(vals_hbm.at[wid, t], valsvm)
        pltpu.sync_copy(pos_hbm.at[wid, t], posvm)

        copies = []
        for b in range(CAP // 128):
            copies.append(pltpu.async_copy(
                valsvm.at[pl.ds(b * 128, 128)], out.at[posvm.at[b]], sem))
        for cp in copies:
            cp.wait()

        pltpu.sync_copy(idx2d.at[pl.ds(wid * 4, 4)], gi2d)

        def tvec(a):
            def tv16(mm):
                iv = gi2d[a, pl.ds(mm * 16, 16)]
                m = iv >= TAIL_START
                n0 = wid * 512 + a * 128 + mm * 16 + _IOTA()
                dump = BATCH + ((n0 + wid) & 127)
                tpos2d[a, pl.ds(mm * 16, 16)] = jnp.where(m, n0, dump)
                gi2d[a, pl.ds(mm * 16, 16)] = jnp.where(
                    m, iv - TAIL_START, n0 & 511)
            for mm in range(8):
                tv16(mm)

        for a in range(4):
            tvec(a)

        copies = []
        for a in range(4):
            copies.append(pltpu.async_copy(
                tail_tbl.at[gi2d.at[a]], ttvm.at[pl.ds(a * 128, 128)], sem))
        for cp in copies:
            cp.wait()
        copies = []
        for a in range(4):
            copies.append(pltpu.async_copy(
                ttvm.at[pl.ds(a * 128, 128)], out.at[tpos2d.at[a]], sem))
        for cp in copies:
            cp.wait()


BM4 = 512
QH = 4 * HIDDEN


def _unpack_lohi(x):
    lo = lax.bitcast_convert_type((x & 0xFFFF).astype(jnp.uint16),
                                  jnp.bfloat16)
    hi = lax.bitcast_convert_type(
        lax.shift_right_logical(x, 16).astype(jnp.uint16), jnp.bfloat16)
    return lo, hi


def _mlp_body(ue_ref, be_ref, w1ue_ref, w1uo_ref, w1be_ref, w1bo_ref,
              b1_ref, w2r_ref, b2_ref, out_ref):
    ulo, uhi = _unpack_lohi(ue_ref[...])
    blo, bhi = _unpack_lohi(be_ref[...])
    h = jnp.dot(ulo, w1ue_ref[...], preferred_element_type=jnp.float32)
    h += jnp.dot(uhi, w1uo_ref[...], preferred_element_type=jnp.float32)
    h += jnp.dot(blo, w1be_ref[...], preferred_element_type=jnp.float32)
    h += jnp.dot(bhi, w1bo_ref[...], preferred_element_type=jnp.float32)
    h = jnp.maximum(h + b1_ref[...], 0.0)
    hw = h * w2r_ref[...]
    segs = [jnp.sum(hw[:, k * HIDDEN:(k + 1) * HIDDEN], axis=1,
                    keepdims=True) for k in range(4)]
    out_ref[...] = jnp.concatenate(segs, axis=1) + b2_ref[...]


def _quad_eo(w_bf):
    z = jnp.zeros_like(w_bf)
    rows = [jnp.concatenate([w_bf if i == k else z for i in range(4)], axis=1)
            for k in range(4)]
    bd = jnp.concatenate(rows, axis=0)
    return bd[0::2], bd[1::2]


def _mlp_tc(uePr, bePr, W1, b1, W2, b2):
    w1ue, w1uo = _quad_eo(W1[:EMBED].astype(jnp.bfloat16))
    w1be, w1bo = _quad_eo(W1[EMBED:].astype(jnp.bfloat16))
    b1t = jnp.tile(b1, 4).reshape(1, QH)
    w2t = jnp.tile(W2.reshape(-1), 4).reshape(1, QH)
    b2r = b2.reshape(1, 1)
    grid = (BATCH // (4 * BM4),)
    out4 = pl.pallas_call(
        _mlp_body,
        grid=grid,
        in_specs=[
            pl.BlockSpec((BM4, 128), lambda i: (i, 0)),
            pl.BlockSpec((BM4, 128), lambda i: (i, 0)),
            pl.BlockSpec((128, QH), lambda i: (0, 0)),
            pl.BlockSpec((128, QH), lambda i: (0, 0)),
            pl.BlockSpec((128, QH), lambda i: (0, 0)),
            pl.BlockSpec((128, QH), lambda i: (0, 0)),
            pl.BlockSpec((1, QH), lambda i: (0, 0)),
            pl.BlockSpec((1, QH), lambda i: (0, 0)),
            pl.BlockSpec((1, 1), lambda i: (0, 0)),
        ],
        out_specs=pl.BlockSpec((BM4, 4), lambda i: (i, 0)),
        out_shape=jax.ShapeDtypeStruct((BATCH // 4, 4), jnp.float32),
    )(uePr, bePr, w1ue, w1uo, w1be, w1bo, b1t, w2t, b2r)
    return out4.reshape(BATCH, 1)


def _pack_tail(tail_f32):
    tb = tail_f32.astype(jnp.bfloat16).reshape(TAIL, WORDS, 2)
    return lax.bitcast_convert_type(tb, jnp.int32).reshape(TAIL, WORDS)


def kernel(users, books, user_table, book_table, W1, b1, W2, b2):
    users_i = users.astype(jnp.int32)
    books_i = books.astype(jnp.int32)
    t3u = user_table.T.reshape(8, 8, V)
    t3b = book_table.T.reshape(8, 8, V)

    vals, pos = _scan_sc(users_i, books_i, t3u, t3b)
    vals2 = vals.reshape(NW, 2, CAP, WORDS)

    tailu = _pack_tail(user_table[TAIL_START:])
    tailb = _pack_tail(book_table[TAIL_START:])
    u2d = users_i.reshape(128, 128)
    b2d = books_i.reshape(128, 128)

    ueP, beP = _scatter_sc(vals2, pos, u2d, b2d, tailu, tailb)

    uePr = ueP.reshape(NOUT * WORDS // 128, 128)
    bePr = beP.reshape(NOUT * WORDS // 128, 128)
    return _mlp_tc(uePr, bePr, W1, b1, W2, b2)
